# Initial kernel scaffold; baseline (speedup 1.0000x reference)
import jax
import jax.numpy as jnp
from jax.experimental import pallas as pl


def kernel(x, params, edge_index, batch):
    raise NotImplementedError("write your pallas kernel here")


# speedup vs baseline: 17.1784x; 17.1784x over previous
import dataclasses
import functools

import jax
import jax.numpy as jnp
from jax import lax
from jax.experimental import pallas as pl
from jax.experimental.pallas import tpu as pltpu
from jax.experimental.pallas import tpu_sc as plsc

N = 10000
E = 320000
D = 128
NGRAPH = 128
NCLS = 10
LAYERS = 3
EPS = 1e-5

NC, NS, LANES = 2, 16, 16
NW = NC * NS
ET = E // NW
ET16 = E // NS
NP = 10240
RPT = NP // NS
ZR = 32
DH = D // 2

KC = 80
NCH16 = ET16 // KC
KA = 40
NCH32 = ET // KA

F32 = jnp.float32
I32 = jnp.int32

_VMESH = plsc.VectorSubcoreMesh(core_axis_name="c", subcore_axis_name="s")

_SC_CP = pltpu.CompilerParams(use_tc_tiling_on_sc=False)
if "needs_layout_passes" in pltpu.CompilerParams.__dataclass_fields__:
    _SC_CP = dataclasses.replace(_SC_CP, needs_layout_passes=False)


def _bcast_lane(vec, lane):
    idx = jnp.full((LANES, 1), lane, I32)
    dnums = lax.GatherDimensionNumbers(
        offset_dims=(), collapsed_slice_dims=(0,), start_index_map=(0,))
    return lax.gather(vec, idx, dnums, (1,),
                      mode=lax.GatherScatterMode.PROMISE_IN_BOUNDS)


def _zero_rows(zbuf, acc, sid, d):
    @pl.loop(0, ZR)
    def _(r):
        @pl.loop(0, d // LANES)
        def _(q):
            zbuf[r, pl.ds(q * LANES, LANES)] = jnp.zeros((LANES,), F32)

    @pl.loop(0, RPT // ZR)
    def _(z):
        pltpu.sync_copy(zbuf, acc.at[pl.ds(sid * RPT + z * ZR, ZR)])


def _hist_body(row_hbm, out_hbm, idx_v, bins_v, sem):
    scid = lax.axis_index("c")
    sid = lax.axis_index("s")
    wid = scid * NS + sid
    pltpu.async_copy(row_hbm.at[wid], idx_v, sem).wait()

    @pl.loop(0, N // LANES)
    def _(i):
        bins_v[pl.ds(i * LANES, LANES)] = jnp.zeros((LANES,), F32)

    ones = jnp.ones((LANES,), F32)

    @pl.loop(0, ET // LANES)
    def _(j):
        idx = idx_v[pl.ds(j * LANES, LANES)]
        plsc.addupdate_scatter(bins_v, [idx], ones)

    pltpu.sync_copy(bins_v, out_hbm.at[wid])


def _sc_hist(row_w):
    return pl.kernel(
        _hist_body,
        out_type=jax.ShapeDtypeStruct((NW, N), F32),
        mesh=_VMESH,
        compiler_params=_SC_CP,
        scratch_types=[
            pltpu.VMEM((ET,), I32),
            pltpu.VMEM((N,), F32),
            pltpu.SemaphoreType.DMA,
        ],
    )(row_w)


def _unw_body(nch, d, xs_hbm, rowr_hbm, colr_hbm, out_hbm,
              idxr_v, idxc_v, buf0, buf1, zbuf, acc, sem0, sem1):
    scid = lax.axis_index("c")
    sid = lax.axis_index("s")

    _zero_rows(zbuf, acc, sid, d)
    pltpu.sync_copy(rowr_hbm.at[scid, sid], idxr_v)
    pltpu.sync_copy(colr_hbm.at[scid, sid], idxc_v)
    plsc.subcore_barrier()

    pltpu.async_copy(xs_hbm.at[idxr_v.at[0]], buf0, sem0)
    pltpu.async_copy(xs_hbm.at[idxr_v.at[1]], buf1, sem1)

    @pl.loop(0, nch, step=2)
    def _(c):
        pltpu.make_async_copy(xs_hbm.at[idxr_v.at[c]], buf0, sem0).wait()
        pltpu.sync_copy(buf0, acc.at[idxc_v.at[c]], add=True)

        @pl.when(c + 2 < nch)
        def _():
            pltpu.async_copy(xs_hbm.at[idxr_v.at[c + 2]], buf0, sem0)

        pltpu.make_async_copy(xs_hbm.at[idxr_v.at[c + 1]], buf1, sem1).wait()
        pltpu.sync_copy(buf1, acc.at[idxc_v.at[c + 1]], add=True)

        @pl.when(c + 3 < nch)
        def _():
            pltpu.async_copy(xs_hbm.at[idxr_v.at[c + 3]], buf1, sem1)

    plsc.subcore_barrier()
    pltpu.sync_copy(acc.at[pl.ds(sid * RPT, RPT)],
                    out_hbm.at[scid, pl.ds(sid * RPT, RPT)])


def _sc_gs(xs_table, rowr, colr, nch, k, d):
    return pl.kernel(
        functools.partial(_unw_body, nch, d),
        out_type=jax.ShapeDtypeStruct((NC, NP, d), F32),
        mesh=_VMESH,
        compiler_params=_SC_CP,
        scratch_types=[
            pltpu.VMEM((nch, k), I32),
            pltpu.VMEM((nch, k), I32),
            pltpu.VMEM((k, d), F32),
            pltpu.VMEM((k, d), F32),
            pltpu.VMEM((ZR, d), F32),
            pltpu.VMEM_SHARED((NP, d), F32),
            pltpu.SemaphoreType.DMA,
            pltpu.SemaphoreType.DMA,
        ],
    )(xs_table, rowr, colr)


def _sc_unw(xs, row, col):
    xs2 = jnp.concatenate([xs[:, :DH], xs[:, DH:]], axis=0)
    rowo = jnp.stack([row, row + N]).reshape(NC, NS, NCH16, KC)
    colo = jnp.broadcast_to(col.reshape(1, NS, NCH16, KC),
                            (NC, NS, NCH16, KC))
    out = _sc_gs(xs2, rowo, colo, NCH16, KC, DH)
    return jnp.concatenate([out[0, :N], out[1, :N]], axis=1)


def _sc_att(xs_att, row, col):
    rowo = row.reshape(NC, NS, NCH32, KA)
    colo = col.reshape(NC, NS, NCH32, KA)
    out = _sc_gs(xs_att, rowo, colo, NCH32, KA, 16)
    return out[0, :N] + out[1, :N]


def _ew_body(row_hbm, col_hbm, u_hbm, v_hbm, ew_hbm, deg_hbm,
             ir_v, ic_v, u_v, v_v, ew_v, bins_v, sem):
    scid = lax.axis_index("c")
    sid = lax.axis_index("s")
    wid = scid * NS + sid
    pltpu.sync_copy(row_hbm.at[wid], ir_v)
    pltpu.sync_copy(col_hbm.at[wid], ic_v)
    pltpu.sync_copy(u_hbm, u_v)
    pltpu.sync_copy(v_hbm, v_v)

    @pl.loop(0, N // LANES)
    def _(i):
        bins_v[pl.ds(i * LANES, LANES)] = jnp.zeros((LANES,), F32)

    @pl.loop(0, ET // LANES)
    def _(j):
        ir = ir_v[pl.ds(j * LANES, LANES)]
        ic = ic_v[pl.ds(j * LANES, LANES)]
        gu = plsc.load_gather(u_v, [ir])
        gv = plsc.load_gather(v_v, [ic])
        ew = 1.0 / (1.0 + jnp.exp(-(gu + gv)))
        ew_v[pl.ds(j * LANES, LANES)] = ew
        plsc.addupdate_scatter(bins_v, [ir], ew)

    pltpu.sync_copy(ew_v, ew_hbm.at[wid])
    pltpu.sync_copy(bins_v, deg_hbm.at[wid])


def _sc_edge_weights(row_w, col_w, u, v):
    return pl.kernel(
        _ew_body,
        out_type=(jax.ShapeDtypeStruct((NW, ET), F32),
                  jax.ShapeDtypeStruct((NW, N), F32)),
        mesh=_VMESH,
        compiler_params=_SC_CP,
        scratch_types=[
            pltpu.VMEM((ET,), I32),
            pltpu.VMEM((ET,), I32),
            pltpu.VMEM((N,), F32),
            pltpu.VMEM((N,), F32),
            pltpu.VMEM((ET,), F32),
            pltpu.VMEM((N,), F32),
            pltpu.SemaphoreType.DMA,
        ],
    )(row_w, col_w, u, v)


def _w_body(half, xs4_hbm, rowo_hbm, col_hbm, u_hbm, v_hbm, out_hbm,
            iro_v, ic_v, u_v, v_v, ew_s, buf0, buf1, zbuf, acc, sem0, sem1):
    scid = lax.axis_index("c")
    sid = lax.axis_index("s")
    sgn = 1.0 - 2.0 * scid.astype(F32)
    off = scid * (2 * N) + half * N

    _zero_rows(zbuf, acc, sid, DH)
    pltpu.sync_copy(rowo_hbm.at[scid, sid], iro_v)
    pltpu.sync_copy(col_hbm.at[sid], ic_v)
    pltpu.sync_copy(u_hbm, u_v)
    pltpu.sync_copy(v_hbm, v_v)
    plsc.subcore_barrier()

    pltpu.async_copy(xs4_hbm.at[iro_v.at[0]], buf0, sem0)
    pltpu.async_copy(xs4_hbm.at[iro_v.at[1]], buf1, sem1)

    def _do_chunk(c, buf, sem):
        pltpu.make_async_copy(xs4_hbm.at[iro_v.at[c]], buf, sem).wait()
        for g in range(KC // LANES):
            iro = iro_v[c, pl.ds(g * LANES, LANES)]
            ir = iro - off
            ic = ic_v[c, pl.ds(g * LANES, LANES)]
            gu = plsc.load_gather(u_v, [ir])
            gv = plsc.load_gather(v_v, [ic])
            ewv = 1.0 / (1.0 + jnp.exp(-(gu + gv) * sgn))
            for jl in range(LANES):
                w = _bcast_lane(ewv, jl)
                j = g * LANES + jl
                for q in range(DH // LANES):
                    sl = (j, pl.ds(q * LANES, LANES))
                    buf[sl] = buf[sl] * w
        pltpu.sync_copy(buf, acc.at[ic_v.at[c]], add=True)

    @pl.loop(0, NCH16, step=2)
    def _(c):
        _do_chunk(c, buf0, sem0)

        @pl.when(c + 2 < NCH16)
        def _():
            pltpu.async_copy(xs4_hbm.at[iro_v.at[c + 2]], buf0, sem0)

        _do_chunk(c + 1, buf1, sem1)

        @pl.when(c + 3 < NCH16)
        def _():
            pltpu.async_copy(xs4_hbm.at[iro_v.at[c + 3]], buf1, sem1)

    plsc.subcore_barrier()
    pltpu.sync_copy(acc.at[pl.ds(sid * RPT, RPT)],
                    out_hbm.at[scid, pl.ds(sid * RPT, RPT)])


def _sc_weighted_half(half, xs4, rowo, col_r, u, v):
    return pl.kernel(
        functools.partial(_w_body, half),
        out_type=jax.ShapeDtypeStruct((NC, NP, DH), F32),
        mesh=_VMESH,
        compiler_params=_SC_CP,
        scratch_types=[
            pltpu.VMEM((NCH16, KC), I32),
            pltpu.VMEM((NCH16, KC), I32),
            pltpu.VMEM((N,), F32),
            pltpu.VMEM((N,), F32),
            pltpu.VMEM((KC,), F32),
            pltpu.VMEM((KC, DH), F32),
            pltpu.VMEM((KC, DH), F32),
            pltpu.VMEM((ZR, DH), F32),
            pltpu.VMEM_SHARED((NP, DH), F32),
            pltpu.SemaphoreType.DMA,
            pltpu.SemaphoreType.DMA,
        ],
    )(xs4, rowo, col_r, u, v)


def _sc_weighted(xs_c, xs_o, row, col, u, v):
    xs4 = jnp.concatenate([xs_c[:, :DH], xs_c[:, DH:],
                           xs_o[:, :DH], xs_o[:, DH:]], axis=0)
    col_r = col.reshape(NS, NCH16, KC)
    S = []
    for half in (0, 1):
        rowo = jnp.stack([row + half * N, row + (2 * N + half * N)])
        rowo = rowo.reshape(NC, NS, NCH16, KC)
        S.append(_sc_weighted_half(half, xs4, rowo, col_r, u, v))
    S_c = jnp.concatenate([S[0][0, :N], S[1][0, :N]], axis=1)
    S_o = jnp.concatenate([S[0][1, :N], S[1][1, :N]], axis=1)
    return S_c, S_o


def _bn(x, g, b):
    mu = jnp.mean(x, axis=0)
    var = jnp.mean((x - mu) ** 2, axis=0)
    return (x - mu) / jnp.sqrt(var + EPS) * g + b


def kernel(x, params, edge_index, batch):
    p = params
    row, col = edge_index[0], edge_index[1]
    row_w = row.reshape(NW, ET)
    col_w = col.reshape(NW, ET)

    hist_p = _sc_hist(row_w)
    hist = jnp.sum(hist_p, axis=0)
    deg = hist + 1.0
    dis = deg ** -0.5

    h = _bn(x, p['bn_feat_g'], p['bn_feat_b'])
    h = jax.nn.relu(h @ p['conv_feat_W'] + p['conv_feat_b'])
    for i in range(LAYERS):
        hb = _bn(h, p[f'bn{i}_g'], p[f'bn{i}_b'])
        xw = hb @ p[f'conv{i}_W']
        xs = dis[:, None] * xw
        S = _sc_unw(xs, row, col)
        h = jax.nn.relu(dis[:, None] * S + dis[:, None] ** 2 * xw
                        + p[f'conv{i}_b'])

    att_xw = h @ p['att_conv_W']
    xs_att = jnp.pad(dis[:, None] * att_xw, ((0, 0), (0, 14)))
    S_att = _sc_att(xs_att, row, col)[:, :2]
    att_out = dis[:, None] * S_att + dis[:, None] ** 2 * att_xw \
        + p['att_conv_b']
    node_att = jax.nn.softmax(att_out, axis=-1)

    Wtop, Wbot = p['att_mlp_W'][:D], p['att_mlp_W'][D:]
    u = h @ (Wtop[:, 0] - Wtop[:, 1])
    v = h @ (Wbot[:, 0] - Wbot[:, 1]) + (p['att_mlp_b'][0] - p['att_mlp_b'][1])
    ew_p, degc_p = _sc_edge_weights(row_w, col_w, u, v)
    segew = jnp.sum(degc_p, axis=0)
    degc = segew + 1.0
    dego = hist - segew + 1.0
    disc = degc ** -0.5
    diso = dego ** -0.5

    def branch_pre(tag, natt, disb):
        xin = natt[:, None] * h
        xb = _bn(xin, p[f'{tag}_bn_g'], p[f'{tag}_bn_b'])
        xw = xb @ p[f'{tag}_conv_W']
        return xw, disb[:, None] * xw

    xw_c, xs_c = branch_pre('xc', node_att[:, 0], disc)
    xw_o, xs_o = branch_pre('xo', node_att[:, 1], diso)
    S_c, S_o = _sc_weighted(xs_c, xs_o, row, col, u, v)

    def branch_post(tag, S, xw, disb):
        full = jax.nn.elu(disb[:, None] * S + disb[:, None] ** 2 * xw
                          + p[f'{tag}_conv_b'])
        return jax.ops.segment_sum(full, batch, num_segments=NGRAPH)

    xc = branch_post('xc', S_c, xw_c, disc)
    xo = branch_post('xo', S_o, xw_o, diso)

    def head(z, tag):
        z = _bn(z, p[f'{tag}_bn1_g'], p[f'{tag}_bn1_b'])
        z = jax.nn.relu(z @ p[f'{tag}_fc1_W'] + p[f'{tag}_fc1_b'])
        z = _bn(z, p[f'{tag}_bn2_g'], p[f'{tag}_bn2_b'])
        z = z @ p[f'{tag}_fc2_W'] + p[f'{tag}_fc2_b']
        return jax.nn.log_softmax(z, axis=-1)

    xc_logis = head(xc, 'c')
    xo_logis = head(xo, 'o')
    z = jnp.concatenate([xc, xo], axis=1)
    z = _bn(z, p['co_bn1_g'], p['co_bn1_b'])
    z = z @ p['co_fc1_W'] + p['co_fc1_b']
    z = jax.nn.elu(jax.nn.elu(z))
    z = _bn(z, p['co_bn2_g'], p['co_bn2_b'])
    z = z @ p['co_fc2_W'] + p['co_fc2_b']
    xco_logis = jax.nn.log_softmax(z, axis=-1)
    return (xc_logis, xo_logis, xco_logis)
